# initial kernel scaffold (unmeasured)
import jax
import jax.numpy as jnp
from jax import lax
from jax.experimental import pallas as pl
from jax.experimental.pallas import tpu as pltpu

N_DEV = 4
BLK = 64


def kernel(x, Wq, K_ext, V_ext, Wo):
    B, Sq, E = x.shape
    Dm = Wq.shape[1]
    _, Skv_loc, Hq, Dh = K_ext.shape

    K2 = K_ext.reshape(B, Skv_loc, Hq * Dh)
    V2 = V_ext.reshape(B, Skv_loc, Hq * Dh)

    def body(x_ref, wq_ref, k_ref, v_ref, wo_ref, out_ref,
             kbuf, vbuf, ksend, krecv, vsend, vrecv):
        my = lax.axis_index("i")
        left = (my + N_DEV - 1) % N_DEV
        right = (my + 1) % N_DEV

        barrier_sem = pltpu.get_barrier_semaphore()
        for nbr in (left, right):
            pl.semaphore_signal(
                barrier_sem, inc=1,
                device_id=(nbr,), device_id_type=pl.DeviceIdType.MESH,
            )
        pl.semaphore_wait(barrier_sem, 2)

        wq = wq_ref[...].astype(jnp.bfloat16)
        Qs = []
        for b in range(B):
            q = lax.dot_general(
                x_ref[b].astype(jnp.bfloat16), wq,
                (((1,), (0,)), ((), ())),
                preferred_element_type=jnp.float32,
            )
            Qs.append((q * 0.125).astype(jnp.bfloat16))

        kbuf[0] = k_ref[...].astype(jnp.bfloat16)
        vbuf[0] = v_ref[...].astype(jnp.bfloat16)

        m = [[jnp.full((Sq, 1), -1e30, jnp.float32) for _ in range(Hq)]
             for _ in range(B)]
        l = [[jnp.zeros((Sq, 1), jnp.float32) for _ in range(Hq)]
             for _ in range(B)]
        acc = [[jnp.zeros((Sq, Dh), jnp.float32) for _ in range(Hq)]
               for _ in range(B)]

        row_blk = lax.broadcasted_iota(jnp.int32, (Sq, Skv_loc), 0) // BLK
        col_loc = lax.broadcasted_iota(jnp.int32, (Sq, Skv_loc), 1) // BLK
        blocks_per_shard = Skv_loc // BLK

        for h_step in range(N_DEV):
            slot = h_step % 2
            if h_step < N_DEV - 1:
                k_rdma = pltpu.make_async_remote_copy(
                    src_ref=kbuf.at[slot], dst_ref=kbuf.at[1 - slot],
                    send_sem=ksend.at[slot], recv_sem=krecv.at[1 - slot],
                    device_id=(right,), device_id_type=pl.DeviceIdType.MESH,
                )
                v_rdma = pltpu.make_async_remote_copy(
                    src_ref=vbuf.at[slot], dst_ref=vbuf.at[1 - slot],
                    send_sem=vsend.at[slot], recv_sem=vrecv.at[1 - slot],
                    device_id=(right,), device_id_type=pl.DeviceIdType.MESH,
                )
                k_rdma.start()
                v_rdma.start()

            origin = (my - h_step) % N_DEV
            col_blk = col_loc + origin * blocks_per_shard
            mask = (
                (row_blk == col_blk)
                | (col_blk == 0)
                | ((row_blk + col_blk) % 3 == 0)
            )

            for b in range(B):
                for hh in range(Hq):
                    qh = Qs[b][:, hh * Dh:(hh + 1) * Dh]
                    kh = kbuf[slot, b, :, hh * Dh:(hh + 1) * Dh]
                    s = lax.dot_general(
                        qh, kh, (((1,), (1,)), ((), ())),
                        preferred_element_type=jnp.float32,
                    )
                    s = jnp.where(mask, s, -1e9)
                    m_new = jnp.maximum(m[b][hh], s.max(axis=-1, keepdims=True))
                    alpha = jnp.exp(m[b][hh] - m_new)
                    p = jnp.exp(s - m_new)
                    vh = vbuf[slot, b, :, hh * Dh:(hh + 1) * Dh]
                    pv = lax.dot_general(
                        p.astype(jnp.bfloat16), vh, (((1,), (0,)), ((), ())),
                        preferred_element_type=jnp.float32,
                    )
                    l[b][hh] = l[b][hh] * alpha + p.sum(axis=-1, keepdims=True)
                    acc[b][hh] = acc[b][hh] * alpha + pv
                    m[b][hh] = m_new

            if h_step < N_DEV - 1:
                k_rdma.wait()
                v_rdma.wait()

        wo = wo_ref[...].astype(jnp.bfloat16)
        for b in range(B):
            ctx = jnp.concatenate(
                [acc[b][hh] / l[b][hh] for hh in range(Hq)], axis=1
            ).astype(jnp.bfloat16)
            out_ref[b] = lax.dot_general(
                ctx, wo, (((1,), (0,)), ((), ())),
                preferred_element_type=jnp.float32,
            )

    return pl.pallas_call(
        body,
        out_shape=jax.ShapeDtypeStruct((B, Sq, E), jnp.float32),
        in_specs=[pl.BlockSpec(memory_space=pltpu.VMEM)] * 5,
        out_specs=pl.BlockSpec(memory_space=pltpu.VMEM),
        scratch_shapes=[
            pltpu.VMEM((2, B, Skv_loc, Hq * Dh), jnp.bfloat16),
            pltpu.VMEM((2, B, Skv_loc, Hq * Dh), jnp.bfloat16),
            pltpu.SemaphoreType.DMA((2,)),
            pltpu.SemaphoreType.DMA((2,)),
            pltpu.SemaphoreType.DMA((2,)),
            pltpu.SemaphoreType.DMA((2,)),
        ],
        compiler_params=pltpu.CompilerParams(collective_id=0),
    )(x, Wq, K2, V2, Wo)


# baseline (device time: 103537 ns/iter reference)
import jax
import jax.numpy as jnp
from jax import lax
from jax.experimental import pallas as pl
from jax.experimental.pallas import tpu as pltpu

N_DEV = 4
BLK = 64


def kernel(x, Wq, K_ext, V_ext, Wo):
    B, Sq, E = x.shape
    Dm = Wq.shape[1]
    _, Skv_loc, Hq, Dh = K_ext.shape

    K2 = K_ext.reshape(B, Skv_loc, Hq * Dh)
    V2 = V_ext.reshape(B, Skv_loc, Hq * Dh)

    def body(x_ref, wq_ref, k_ref, v_ref, wo_ref, out_ref,
             kbuf, vbuf, ksend, krecv, vsend, vrecv):
        my = lax.axis_index("i")
        left = (my + N_DEV - 1) % N_DEV
        right = (my + 1) % N_DEV

        barrier_sem = pltpu.get_barrier_semaphore()
        for nbr in (left, right):
            pl.semaphore_signal(
                barrier_sem, inc=1,
                device_id=(nbr,), device_id_type=pl.DeviceIdType.MESH,
            )
        pl.semaphore_wait(barrier_sem, 2)

        wq = wq_ref[...].astype(jnp.bfloat16)
        Qs = []
        for b in range(B):
            q = lax.dot_general(
                x_ref[b].astype(jnp.bfloat16), wq,
                (((1,), (0,)), ((), ())),
                preferred_element_type=jnp.float32,
            )
            Qs.append((q * 0.125).astype(jnp.bfloat16))

        kbuf[0] = k_ref[...].astype(jnp.bfloat16)
        vbuf[0] = v_ref[...].astype(jnp.bfloat16)

        m = [[jnp.full((Sq, 1), -1e30, jnp.float32) for _ in range(Hq)]
             for _ in range(B)]
        l = [[jnp.zeros((Sq, 1), jnp.float32) for _ in range(Hq)]
             for _ in range(B)]
        acc = [[jnp.zeros((Sq, Dh), jnp.float32) for _ in range(Hq)]
               for _ in range(B)]

        row_blk = lax.broadcasted_iota(jnp.int32, (Sq, Skv_loc), 0) // BLK
        col_loc = lax.broadcasted_iota(jnp.int32, (Sq, Skv_loc), 1) // BLK
        blocks_per_shard = Skv_loc // BLK

        for h_step in range(N_DEV):
            slot = h_step % 2
            if h_step < N_DEV - 1:
                k_rdma = pltpu.make_async_remote_copy(
                    src_ref=kbuf.at[slot], dst_ref=kbuf.at[1 - slot],
                    send_sem=ksend.at[slot], recv_sem=krecv.at[1 - slot],
                    device_id=(right,), device_id_type=pl.DeviceIdType.MESH,
                )
                v_rdma = pltpu.make_async_remote_copy(
                    src_ref=vbuf.at[slot], dst_ref=vbuf.at[1 - slot],
                    send_sem=vsend.at[slot], recv_sem=vrecv.at[1 - slot],
                    device_id=(right,), device_id_type=pl.DeviceIdType.MESH,
                )
                k_rdma.start()
                v_rdma.start()

            origin = (my - h_step) % N_DEV
            col_blk = col_loc + origin * blocks_per_shard
            mask = (
                (row_blk == col_blk)
                | (col_blk == 0)
                | ((row_blk + col_blk) % 3 == 0)
            )

            for b in range(B):
                for hh in range(Hq):
                    qh = Qs[b][:, hh * Dh:(hh + 1) * Dh]
                    kh = kbuf[slot, b, :, hh * Dh:(hh + 1) * Dh]
                    s = lax.dot_general(
                        qh, kh, (((1,), (1,)), ((), ())),
                        preferred_element_type=jnp.float32,
                    )
                    s = jnp.where(mask, s, -1e9)
                    m_new = jnp.maximum(m[b][hh], s.max(axis=-1, keepdims=True))
                    alpha = jnp.exp(m[b][hh] - m_new)
                    p = jnp.exp(s - m_new)
                    vh = vbuf[slot, b, :, hh * Dh:(hh + 1) * Dh]
                    pv = lax.dot_general(
                        p.astype(jnp.bfloat16), vh, (((1,), (0,)), ((), ())),
                        preferred_element_type=jnp.float32,
                    )
                    l[b][hh] = l[b][hh] * alpha + p.sum(axis=-1, keepdims=True)
                    acc[b][hh] = acc[b][hh] * alpha + pv
                    m[b][hh] = m_new

            if h_step < N_DEV - 1:
                k_rdma.wait()
                v_rdma.wait()

        wo = wo_ref[...].astype(jnp.bfloat16)
        for b in range(B):
            ctx = jnp.concatenate(
                [acc[b][hh] / l[b][hh] for hh in range(Hq)], axis=1
            ).astype(jnp.bfloat16)
            out_ref[b] = lax.dot_general(
                ctx, wo, (((1,), (0,)), ((), ())),
                preferred_element_type=jnp.float32,
            )

    return pl.pallas_call(
        body,
        out_shape=jax.ShapeDtypeStruct((B, Sq, E), jnp.float32),
        in_specs=[pl.BlockSpec(memory_space=pltpu.VMEM)] * 5,
        out_specs=pl.BlockSpec(memory_space=pltpu.VMEM),
        scratch_shapes=[
            pltpu.VMEM((2, B, Skv_loc, Hq * Dh), jnp.bfloat16),
            pltpu.VMEM((2, B, Skv_loc, Hq * Dh), jnp.bfloat16),
            pltpu.SemaphoreType.DMA((2,)),
            pltpu.SemaphoreType.DMA((2,)),
            pltpu.SemaphoreType.DMA((2,)),
            pltpu.SemaphoreType.DMA((2,)),
        ],
        compiler_params=pltpu.CompilerParams(
            collective_id=0, vmem_limit_bytes=100 * 1024 * 1024
        ),
    )(x, Wq, K2, V2, Wo)


# device time: 47392 ns/iter; 2.1847x vs baseline; 2.1847x over previous
import jax
import jax.numpy as jnp
from jax import lax
from jax.experimental import pallas as pl
from jax.experimental.pallas import tpu as pltpu

N_DEV = 4
BLK = 64


def kernel(x, Wq, K_ext, V_ext, Wo):
    B, Sq, E = x.shape
    Dm = Wq.shape[1]
    _, Skv_loc, Hq, Dh = K_ext.shape
    HD = Hq * Dh
    ROWS = HD + Hq

    K2 = K_ext.reshape(B, Skv_loc, HD)
    V2 = V_ext.reshape(B, Skv_loc, HD)

    def body(x_ref, wq_ref, k_ref, v_ref, wo_ref, out_ref,
             abuf, send_sems, recv_sems):
        my = lax.axis_index("i")
        left = (my + N_DEV - 1) % N_DEV
        right = (my + 1) % N_DEV

        barrier_sem = pltpu.get_barrier_semaphore()
        for nbr in (left, right):
            pl.semaphore_signal(
                barrier_sem, inc=1,
                device_id=(nbr,), device_id_type=pl.DeviceIdType.MESH,
            )
        pl.semaphore_wait(barrier_sem, 2)

        wq = wq_ref[...].astype(jnp.bfloat16)
        Qs = []
        for b in range(B):
            q = lax.dot_general(
                x_ref[b].astype(jnp.bfloat16), wq,
                (((1,), (0,)), ((), ())),
                preferred_element_type=jnp.float32,
            )
            Qs.append((q * 0.125).astype(jnp.bfloat16))

        kv_blk = lax.broadcasted_iota(jnp.int32, (Skv_loc, Sq), 0) // BLK
        kv_blk = kv_blk + my * (Skv_loc // BLK)
        q_blk = lax.broadcasted_iota(jnp.int32, (Skv_loc, Sq), 1) // BLK
        maskT = (q_blk == kv_blk) | (kv_blk == 0) | ((q_blk + kv_blk) % 3 == 0)

        glob = []
        for b in range(B):
            parts = []
            lrows = []
            for h in range(Hq):
                qh = Qs[b][:, h * Dh:(h + 1) * Dh]
                kh = k_ref[b, :, h * Dh:(h + 1) * Dh].astype(jnp.bfloat16)
                sT = lax.dot_general(
                    kh, qh, (((1,), (1,)), ((), ())),
                    preferred_element_type=jnp.float32,
                )
                pT = jnp.exp(jnp.where(maskT, sT, -1e30))
                lT = pT.sum(axis=0, keepdims=True)
                vh = v_ref[b, :, h * Dh:(h + 1) * Dh].astype(jnp.bfloat16)
                accT = lax.dot_general(
                    vh, pT.astype(jnp.bfloat16),
                    (((0,), (0,)), ((), ())),
                    preferred_element_type=jnp.float32,
                )
                parts.append(accT)
                lrows.append(lT)
            gb = jnp.concatenate(parts + lrows, axis=0)
            glob.append(gb)
            abuf[0, b] = gb.astype(jnp.bfloat16)

        send_r0 = pltpu.make_async_remote_copy(
            src_ref=abuf.at[0], dst_ref=abuf.at[1],
            send_sem=send_sems.at[0], recv_sem=recv_sems.at[1],
            device_id=(right,), device_id_type=pl.DeviceIdType.MESH,
        )
        send_l0 = pltpu.make_async_remote_copy(
            src_ref=abuf.at[0], dst_ref=abuf.at[2],
            send_sem=send_sems.at[1], recv_sem=recv_sems.at[2],
            device_id=(left,), device_id_type=pl.DeviceIdType.MESH,
        )
        send_r0.start()
        send_l0.start()

        recv_1 = pltpu.make_async_remote_copy(
            src_ref=abuf.at[0], dst_ref=abuf.at[1],
            send_sem=send_sems.at[0], recv_sem=recv_sems.at[1],
            device_id=(left,), device_id_type=pl.DeviceIdType.MESH,
        )
        recv_2 = pltpu.make_async_remote_copy(
            src_ref=abuf.at[0], dst_ref=abuf.at[2],
            send_sem=send_sems.at[1], recv_sem=recv_sems.at[2],
            device_id=(right,), device_id_type=pl.DeviceIdType.MESH,
        )
        recv_1.wait_recv()
        recv_2.wait_recv()

        fwd_r = pltpu.make_async_remote_copy(
            src_ref=abuf.at[1, 0], dst_ref=abuf.at[3, 0],
            send_sem=send_sems.at[2], recv_sem=recv_sems.at[3],
            device_id=(right,), device_id_type=pl.DeviceIdType.MESH,
        )
        fwd_l = pltpu.make_async_remote_copy(
            src_ref=abuf.at[2, 1], dst_ref=abuf.at[3, 1],
            send_sem=send_sems.at[3], recv_sem=recv_sems.at[4],
            device_id=(left,), device_id_type=pl.DeviceIdType.MESH,
        )
        fwd_r.start()
        fwd_l.start()

        for b in range(B):
            glob[b] = glob[b] + abuf[1, b].astype(jnp.float32) \
                              + abuf[2, b].astype(jnp.float32)

        recv_3a = pltpu.make_async_remote_copy(
            src_ref=abuf.at[1, 0], dst_ref=abuf.at[3, 0],
            send_sem=send_sems.at[2], recv_sem=recv_sems.at[3],
            device_id=(left,), device_id_type=pl.DeviceIdType.MESH,
        )
        recv_3b = pltpu.make_async_remote_copy(
            src_ref=abuf.at[2, 1], dst_ref=abuf.at[3, 1],
            send_sem=send_sems.at[3], recv_sem=recv_sems.at[4],
            device_id=(right,), device_id_type=pl.DeviceIdType.MESH,
        )
        recv_3a.wait_recv()
        recv_3b.wait_recv()
        for b in range(B):
            glob[b] = glob[b] + abuf[3, b].astype(jnp.float32)

        wo = wo_ref[...].astype(jnp.bfloat16)
        for b in range(B):
            accT = glob[b]
            ctx_rows = []
            for h in range(Hq):
                recip = 1.0 / accT[HD + h:HD + h + 1, :]
                ctx_rows.append(accT[h * Dh:(h + 1) * Dh, :] * recip)
            ctxT = jnp.concatenate(ctx_rows, axis=0).astype(jnp.bfloat16)
            out_ref[b] = lax.dot_general(
                ctxT, wo, (((0,), (0,)), ((), ())),
                preferred_element_type=jnp.float32,
            )

        send_r0.wait_send()
        send_l0.wait_send()
        fwd_r.wait_send()
        fwd_l.wait_send()

    return pl.pallas_call(
        body,
        out_shape=jax.ShapeDtypeStruct((B, Sq, E), jnp.float32),
        in_specs=[pl.BlockSpec(memory_space=pltpu.VMEM)] * 5,
        out_specs=pl.BlockSpec(memory_space=pltpu.VMEM),
        scratch_shapes=[
            pltpu.VMEM((4, B, ROWS, Sq), jnp.bfloat16),
            pltpu.SemaphoreType.DMA((4,)),
            pltpu.SemaphoreType.DMA((5,)),
        ],
        compiler_params=pltpu.CompilerParams(
            collective_id=0, vmem_limit_bytes=100 * 1024 * 1024
        ),
    )(x, Wq, K2, V2, Wo)
